# baseline (speedup 1.0000x reference)
import jax
import jax.numpy as jnp
from jax.experimental import pallas as pl
from jax.experimental.pallas import tpu as pltpu

N = 128
H = 64
C = 16
CH = 8
NCH = N // C

_MM_L = (((1,), (0,)), ((), ()))
_MM_NT = (((1,), (1,)), ((), ()))
_MM_TT0 = (((0,), (0,)), ((), ()))


def _mpn_body(fa_ref, fb_ref, adj_ref, Wa_ref, Wb_ref, W0_ref,
              W1_ref, Wo_ref, bo_ref, ah_ref, mb_ref,
              ib_ref, ibT_ref, mT_ref, adjT_ref, ia_ref, ma_ref, st_ref,
              fbbuf_ref, fbsem):
    f32 = jnp.float32
    g = pl.program_id(0)
    ng = pl.num_programs(0)
    slot = jax.lax.rem(g, 2)

    def fb_copy(gi, sl):
        return pltpu.make_async_copy(
            fb_ref.at[gi], fbbuf_ref.at[sl], fbsem.at[sl])

    @pl.when(g == 0)
    def _():
        fb_copy(0, 0).start()
        fb_copy(1, 1).start()

    adj = adj_ref[0]
    adjT_ref[...] = adj.T

    ia = jax.nn.relu(jax.lax.dot_general(Wa_ref[...], fa_ref[0], _MM_NT,
                                         preferred_element_type=f32))
    ia_ref[...] = ia

    fb_copy(g, slot).wait()

    def p1(ci, carry):
        A0, X0 = carry
        r = pl.multiple_of(ci * C, C)
        masked = fbbuf_ref[slot, :, pl.ds(r, C), :] * adj_ref[0, pl.ds(r, C)][None, :, :]
        ibc = jax.nn.relu(jax.lax.dot_general(Wb_ref[...], masked, _MM_L,
                                              preferred_element_type=f32))
        ib_ref[:, pl.ds(r, C), :] = ibc
        return A0 + jnp.sum(ibc, axis=1), jnp.maximum(X0, jnp.max(ibc, axis=1))

    A0 = jnp.zeros((H, N), f32)
    X0 = jnp.full((H, N), -jnp.inf, f32)
    A0, X0 = jax.lax.fori_loop(0, NCH, p1, (A0, X0))

    @pl.when(g + 2 < ng)
    def _():
        fb_copy(g + 2, slot).start()

    def tr(hi, _):
        r = pl.multiple_of(hi * CH, CH)
        ibT_ref[pl.ds(r, CH)] = jnp.swapaxes(ib_ref[pl.ds(r, CH)], 1, 2)
        return 0

    jax.lax.fori_loop(0, H // CH, tr, 0)

    ma = ia + A0 * jax.nn.sigmoid(X0)
    ma_ref[...] = ma

    Gm = jax.lax.dot_general(ma, ma, _MM_TT0, preferred_element_type=f32)
    RT = Gm * adjT_ref[...]
    STm = jnp.max(RT, axis=1, keepdims=True)
    STe = jnp.exp(RT - STm)
    st_ref[...] = STe / jnp.sum(STe, axis=1, keepdims=True)

    sums, maxs = [], []
    for ci in range(NCH):
        r = ci * C
        adjc = adj_ref[0, pl.ds(r, C)]
        D = adjc[None, :, :] * ma[:, None, :] - ib_ref[:, pl.ds(r, C), :]
        mm = jax.lax.dot_general(W0_ref[...], D, _MM_L, preferred_element_type=f32)
        out = jax.nn.relu(ibT_ref[:, pl.ds(r, C), :] + mm) * st_ref[pl.ds(r, C)][None, :, :]
        mT_ref[:, pl.ds(r, C), :] = out
        sums.append(jnp.sum(out, axis=2))
        maxs.append(jnp.max(out, axis=2))

    A1 = jnp.concatenate(sums, axis=1)
    X1 = jnp.concatenate(maxs, axis=1)
    ma2 = ma + A1 * jax.nn.sigmoid(X1)
    ma_ref[...] = ma2

    G2 = jax.lax.dot_general(ma2, ma2, _MM_TT0, preferred_element_type=f32)
    R2 = G2 * adj_ref[0]
    S2m = jnp.max(R2, axis=0, keepdims=True)
    S2e = jnp.exp(R2 - S2m)
    st_ref[...] = S2e / jnp.sum(S2e, axis=0, keepdims=True)

    A2 = jnp.zeros((H, N), f32)
    X2 = jnp.full((H, N), -jnp.inf, f32)
    for ci in range(NCH):
        r = ci * C
        adjTc = adjT_ref[pl.ds(r, C)]
        D = adjTc[None, :, :] * ma2[:, r:r + C][:, :, None] - mT_ref[:, pl.ds(r, C), :]
        mm = jax.lax.dot_general(W1_ref[...], D, _MM_L, preferred_element_type=f32)
        out = jax.nn.relu(ib_ref[:, pl.ds(r, C), :] + mm) * st_ref[pl.ds(r, C)][None, :, :]
        mb_ref[0, pl.ds(r, C)] = jnp.swapaxes(out, 0, 1)
        A2 = A2 + jnp.sum(out, axis=1)
        X2 = jnp.maximum(X2, jnp.max(out, axis=1))

    agg2 = A2 * jax.nn.sigmoid(X2)

    Wo = Wo_ref[...]
    pre_o = (jax.lax.dot_general(Wo[:, 0:H], agg2, _MM_L, preferred_element_type=f32)
             + jax.lax.dot_general(Wo[:, H:2 * H], ma_ref[...], _MM_L,
                                   preferred_element_type=f32)
             + jax.lax.dot_general(Wo[:, 2 * H:3 * H], ia_ref[...], _MM_L,
                                   preferred_element_type=f32)
             + jnp.swapaxes(bo_ref[...], 0, 1))
    ah_ref[0] = jax.nn.relu(pre_o)


@jax.jit
def kernel(f_atoms, f_bonds, adj, W_i_atom, W_i_bond, W_h_0, W_h_1, W_o, b_o):
    B, S, n, AF = f_atoms.shape
    BF = f_bonds.shape[-1]
    G = B * S
    fa = f_atoms.reshape(G, n, AF)
    fbT = jnp.transpose(f_bonds, (0, 1, 4, 2, 3)).reshape(G, BF, n, n)
    a = adj.reshape(G, n, n)
    ah, mb = pl.pallas_call(
        _mpn_body,
        grid=(G,),
        in_specs=[
            pl.BlockSpec((1, N, AF), lambda g: (g, 0, 0)),
            pl.BlockSpec(memory_space=pltpu.MemorySpace.HBM),
            pl.BlockSpec((1, N, N), lambda g: (g, 0, 0)),
            pl.BlockSpec((H, AF), lambda g: (0, 0)),
            pl.BlockSpec((H, BF), lambda g: (0, 0)),
            pl.BlockSpec((H, H), lambda g: (0, 0)),
            pl.BlockSpec((H, H), lambda g: (0, 0)),
            pl.BlockSpec((H, 3 * H), lambda g: (0, 0)),
            pl.BlockSpec((1, H), lambda g: (0, 0)),
        ],
        out_specs=[
            pl.BlockSpec((1, H, N), lambda g: (g, 0, 0)),
            pl.BlockSpec((1, N, H, N), lambda g: (g, 0, 0, 0)),
        ],
        out_shape=[
            jax.ShapeDtypeStruct((G, H, N), jnp.float32),
            jax.ShapeDtypeStruct((G, N, H, N), jnp.float32),
        ],
        scratch_shapes=[
            pltpu.VMEM((H, N, N), jnp.float32),
            pltpu.VMEM((H, N, N), jnp.float32),
            pltpu.VMEM((H, N, N), jnp.float32),
            pltpu.VMEM((N, N), jnp.float32),
            pltpu.VMEM((H, N), jnp.float32),
            pltpu.VMEM((H, N), jnp.float32),
            pltpu.VMEM((N, N), jnp.float32),
            pltpu.VMEM((2, BF, N, N), jnp.float32),
            pltpu.SemaphoreType.DMA((2,)),
        ],
    )(fa, fbT, a, W_i_atom, W_i_bond, W_h_0, W_h_1, W_o, b_o.reshape(1, H))
    ah_l = jnp.transpose(ah, (0, 2, 1)).reshape(B, S, n, H)
    mb_l = jnp.transpose(mb, (0, 1, 3, 2)).reshape(B, S, n, n, H)
    return ah_l, mb_l


# speedup vs baseline: 1.1491x; 1.1491x over previous
import jax
import jax.numpy as jnp
from jax.experimental import pallas as pl
from jax.experimental.pallas import tpu as pltpu

N = 128
H = 64
C = 16
CH = 8
NCH = N // C

_MM_L = (((1,), (0,)), ((), ()))
_MM_NT = (((1,), (1,)), ((), ()))
_MM_TT0 = (((0,), (0,)), ((), ()))


def _mpn_body(fa_ref, fb_ref, adj_ref, Wa_ref, Wb_ref, W0_ref,
              W1_ref, Wo_ref, bo_ref, ah_ref, mb_ref,
              ib_ref, ibT_ref, mT_ref, adjT_ref, ia_ref, ma_ref, st_ref,
              fbbuf_ref, fbsem):
    f32 = jnp.float32
    g = pl.program_id(0)
    ng = pl.num_programs(0)
    slot = jax.lax.rem(g, 2)

    def fb_copy(gi, sl):
        return pltpu.make_async_copy(
            fb_ref.at[gi], fbbuf_ref.at[sl], fbsem.at[sl])

    @pl.when(g == 0)
    def _():
        fb_copy(0, 0).start()
        fb_copy(1, 1).start()

    adj = adj_ref[0]
    adjT_ref[...] = adj.T

    ia = jax.nn.relu(jax.lax.dot_general(Wa_ref[...], fa_ref[0], _MM_NT,
                                         preferred_element_type=f32))
    ia_ref[...] = ia

    fb_copy(g, slot).wait()

    A0 = jnp.zeros((H, N), f32)
    X0 = jnp.full((H, N), -jnp.inf, f32)
    for ci in range(NCH):
        r = ci * C
        masked = fbbuf_ref[slot, :, pl.ds(r, C), :] * adj_ref[0, pl.ds(r, C)][None, :, :]
        ibc = jax.nn.relu(jax.lax.dot_general(Wb_ref[...], masked, _MM_L,
                                              preferred_element_type=f32))
        ib_ref[:, pl.ds(r, C), :] = ibc
        A0 = A0 + jnp.sum(ibc, axis=1)
        X0 = jnp.maximum(X0, jnp.max(ibc, axis=1))

    @pl.when(g + 2 < ng)
    def _():
        fb_copy(g + 2, slot).start()

    for hi in range(H // CH):
        r = hi * CH
        ibT_ref[pl.ds(r, CH)] = jnp.swapaxes(ib_ref[pl.ds(r, CH)], 1, 2)

    ma = ia + A0 * jax.nn.sigmoid(X0)
    ma_ref[...] = ma

    Gm = jax.lax.dot_general(ma, ma, _MM_TT0, preferred_element_type=f32)
    RT = Gm * adjT_ref[...]
    STm = jnp.max(RT, axis=1, keepdims=True)
    STe = jnp.exp(RT - STm)
    st_ref[...] = STe / jnp.sum(STe, axis=1, keepdims=True)

    sums, maxs = [], []
    for ci in range(NCH):
        r = ci * C
        adjc = adj_ref[0, pl.ds(r, C)]
        D = adjc[None, :, :] * ma[:, None, :] - ib_ref[:, pl.ds(r, C), :]
        mm = jax.lax.dot_general(W0_ref[...], D, _MM_L, preferred_element_type=f32)
        out = jax.nn.relu(ibT_ref[:, pl.ds(r, C), :] + mm) * st_ref[pl.ds(r, C)][None, :, :]
        mT_ref[:, pl.ds(r, C), :] = out
        sums.append(jnp.sum(out, axis=2))
        maxs.append(jnp.max(out, axis=2))

    A1 = jnp.concatenate(sums, axis=1)
    X1 = jnp.concatenate(maxs, axis=1)
    ma2 = ma + A1 * jax.nn.sigmoid(X1)
    ma_ref[...] = ma2

    G2 = jax.lax.dot_general(ma2, ma2, _MM_TT0, preferred_element_type=f32)
    R2 = G2 * adj_ref[0]
    S2m = jnp.max(R2, axis=0, keepdims=True)
    S2e = jnp.exp(R2 - S2m)
    st_ref[...] = S2e / jnp.sum(S2e, axis=0, keepdims=True)

    A2 = jnp.zeros((H, N), f32)
    X2 = jnp.full((H, N), -jnp.inf, f32)
    for ci in range(NCH):
        r = ci * C
        adjTc = adjT_ref[pl.ds(r, C)]
        D = adjTc[None, :, :] * ma2[:, r:r + C][:, :, None] - mT_ref[:, pl.ds(r, C), :]
        mm = jax.lax.dot_general(W1_ref[...], D, _MM_L, preferred_element_type=f32)
        out = jax.nn.relu(ib_ref[:, pl.ds(r, C), :] + mm) * st_ref[pl.ds(r, C)][None, :, :]
        mb_ref[0, pl.ds(r, C)] = jnp.swapaxes(out, 0, 1)
        A2 = A2 + jnp.sum(out, axis=1)
        X2 = jnp.maximum(X2, jnp.max(out, axis=1))

    agg2 = A2 * jax.nn.sigmoid(X2)

    Wo = Wo_ref[...]
    pre_o = (jax.lax.dot_general(Wo[:, 0:H], agg2, _MM_L, preferred_element_type=f32)
             + jax.lax.dot_general(Wo[:, H:2 * H], ma_ref[...], _MM_L,
                                   preferred_element_type=f32)
             + jax.lax.dot_general(Wo[:, 2 * H:3 * H], ia_ref[...], _MM_L,
                                   preferred_element_type=f32)
             + jnp.swapaxes(bo_ref[...], 0, 1))
    ah_ref[0] = jax.nn.relu(pre_o)


@jax.jit
def kernel(f_atoms, f_bonds, adj, W_i_atom, W_i_bond, W_h_0, W_h_1, W_o, b_o):
    B, S, n, AF = f_atoms.shape
    BF = f_bonds.shape[-1]
    G = B * S
    fa = f_atoms.reshape(G, n, AF)
    fbT = jnp.transpose(f_bonds, (0, 1, 4, 2, 3)).reshape(G, BF, n, n)
    a = adj.reshape(G, n, n)
    ah, mb = pl.pallas_call(
        _mpn_body,
        grid=(G,),
        in_specs=[
            pl.BlockSpec((1, N, AF), lambda g: (g, 0, 0)),
            pl.BlockSpec(memory_space=pltpu.MemorySpace.HBM),
            pl.BlockSpec((1, N, N), lambda g: (g, 0, 0)),
            pl.BlockSpec((H, AF), lambda g: (0, 0)),
            pl.BlockSpec((H, BF), lambda g: (0, 0)),
            pl.BlockSpec((H, H), lambda g: (0, 0)),
            pl.BlockSpec((H, H), lambda g: (0, 0)),
            pl.BlockSpec((H, 3 * H), lambda g: (0, 0)),
            pl.BlockSpec((1, H), lambda g: (0, 0)),
        ],
        out_specs=[
            pl.BlockSpec((1, H, N), lambda g: (g, 0, 0)),
            pl.BlockSpec((1, N, H, N), lambda g: (g, 0, 0, 0)),
        ],
        out_shape=[
            jax.ShapeDtypeStruct((G, H, N), jnp.float32),
            jax.ShapeDtypeStruct((G, N, H, N), jnp.float32),
        ],
        scratch_shapes=[
            pltpu.VMEM((H, N, N), jnp.float32),
            pltpu.VMEM((H, N, N), jnp.float32),
            pltpu.VMEM((H, N, N), jnp.float32),
            pltpu.VMEM((N, N), jnp.float32),
            pltpu.VMEM((H, N), jnp.float32),
            pltpu.VMEM((H, N), jnp.float32),
            pltpu.VMEM((N, N), jnp.float32),
            pltpu.VMEM((2, BF, N, N), jnp.float32),
            pltpu.SemaphoreType.DMA((2,)),
        ],
    )(fa, fbT, a, W_i_atom, W_i_bond, W_h_0, W_h_1, W_o, b_o.reshape(1, H))
    ah_l = jnp.transpose(ah, (0, 2, 1)).reshape(B, S, n, H)
    mb_l = jnp.transpose(mb, (0, 1, 3, 2)).reshape(B, S, n, n, H)
    return ah_l, mb_l
